# baseline (speedup 1.0000x reference)
import functools

import jax
import jax.numpy as jnp
import numpy as np
from jax.experimental import pallas as pl
from jax.experimental.pallas import tpu as pltpu

_B = 4
_T = 48
_N = 300
_C = 33
_E = 128
_NN = _N * _C
_NA = 5000
_K = 8
_S = 2.0
_SIG = 64

_NNP = 9984
_NAP = 5120
_RT = 128
_ACH = 40

_t = np.arange(_T)
_c = np.arange(_C)
_ang = 2.0 * np.pi * np.outer(_c, _t) / _SIG
_DCT_R = (np.cos(_ang) / np.sqrt(_SIG)).astype(np.float32)
_DCT_I = (-np.sin(_ang) / np.sqrt(_SIG)).astype(np.float32)
_w = np.full(_C, 2.0); _w[0] = 1.0; _w[_C - 1] = 1.0
_ang2 = 2.0 * np.pi * np.outer(_t, _c) / _SIG
_IDFT_R = (_w * np.cos(_ang2) / np.sqrt(_SIG)).astype(np.float32)
_IDFT_I = (-_w * np.sin(_ang2) / np.sqrt(_SIG)).astype(np.float32)

_PERMS = np.stack([np.asarray(jax.random.permutation(
    jax.random.fold_in(jax.random.key(42), b), _NN)[:_NA]) for b in range(_B)])
_PERMS_PAD = np.concatenate(
    [_PERMS, np.zeros((_B, _NAP - _NA), np.int64)], axis=1).astype(np.int32)

_COLH = np.eye(_C, dtype=np.float32)[np.arange(_NN) % _C]


def _knn_body(fxb_ref, fyb_ref, sqf_ref, axb_ref, ayb_ref, sqa_ref,
              out_ref, d2_s, cmin_s):
    fxb = fxb_ref[0, 0]
    fyb = fyb_ref[0, 0]
    sqf = sqf_ref[0, 0]
    iota0 = jax.lax.broadcasted_iota(jnp.int32, (_RT, _RT), 0)
    big_i = jnp.int32(2 ** 30)
    inf = jnp.float32(jnp.inf)

    def fill(c, _):
        sl = pl.ds(c * _RT, _RT)
        a_x = axb_ref[0, sl, :]
        a_y = ayb_ref[0, sl, :]
        s_a = sqa_ref[0, sl, :]
        mm = a_x * fxb + a_y * fyb
        ch = (s_a + sqf) - 2.0 * mm
        d2_s[sl, :] = ch
        cmin_s[pl.ds(c, 1), :] = jnp.min(ch, axis=0, keepdims=True)
        return 0

    jax.lax.fori_loop(0, _ACH, fill, 0)

    for k in range(_K):
        m = jnp.min(cmin_s[...], axis=0, keepdims=True)

        def scan(c, am):
            ch = d2_s[pl.ds(c * _RT, _RT), :]
            cand = jnp.where(ch == m, c * _RT + iota0, big_i)
            return jnp.minimum(am, jnp.min(cand, axis=0, keepdims=True))

        am = jax.lax.fori_loop(0, _ACH, scan,
                               jnp.full((1, _RT), big_i, jnp.int32))

        def mask(c, _):
            sl = pl.ds(c * _RT, _RT)
            ch = d2_s[sl, :]
            ch2 = jnp.where((c * _RT + iota0) == am, inf, ch)
            d2_s[sl, :] = ch2
            cmin_s[pl.ds(c, 1), :] = jnp.min(ch2, axis=0, keepdims=True)
            return 0

        jax.lax.fori_loop(0, _ACH, mask, 0)
        out_ref[0, k, :] = am[0]


@functools.partial(jax.jit, static_argnames=("interpret",))
def _knn_topk(fxb, fyb, sqf, axb, ayb, sqa, interpret=False):
    grid = (_B, _NNP // _RT)
    fxb = fxb.reshape(_B, _NNP // _RT, 1, _RT)
    fyb = fyb.reshape(_B, _NNP // _RT, 1, _RT)
    sqf = sqf.reshape(_B, _NNP // _RT, 1, _RT)
    return pl.pallas_call(
        _knn_body,
        grid=grid,
        in_specs=[
            pl.BlockSpec((1, 1, 1, _RT), lambda b, i: (b, i, 0, 0)),
            pl.BlockSpec((1, 1, 1, _RT), lambda b, i: (b, i, 0, 0)),
            pl.BlockSpec((1, 1, 1, _RT), lambda b, i: (b, i, 0, 0)),
            pl.BlockSpec((1, _NAP, 1), lambda b, i: (b, 0, 0)),
            pl.BlockSpec((1, _NAP, 1), lambda b, i: (b, 0, 0)),
            pl.BlockSpec((1, _NAP, 1), lambda b, i: (b, 0, 0)),
        ],
        out_specs=pl.BlockSpec((1, _K, _RT), lambda b, i: (b, 0, i)),
        out_shape=jax.ShapeDtypeStruct((_B, _K, _NNP), jnp.int32),
        scratch_shapes=[
            pltpu.VMEM((_NAP, _RT), jnp.float32),
            pltpu.VMEM((_ACH, _RT), jnp.float32),
        ],
        interpret=interpret,
    )(fxb, fyb, sqf, axb, ayb, sqa)


def _bf(v):
    return v.astype(jnp.bfloat16).astype(jnp.float32)


def kernel(x, theta, Wr_f, Wi_f, Wr_o, Wi_o, g1, b1, w1, bw1, g2, b2, w2, bw2,
           wt, bt, w3, b3, freq_emb, approx):
    xp = jnp.concatenate([x[:, :1, :], x, x[:, -1:, :]], axis=1)
    trend = (xp[:, :-2, :] + xp[:, 1:-1, :] + xp[:, 2:, :]) / 3.0
    seasonal = x - trend

    Sf = jnp.fft.rfft(seasonal, n=_SIG, axis=1, norm='ortho')
    S_perm = jnp.transpose(Sf, (0, 2, 1))
    sr = jnp.real(S_perm).reshape(_B, _NN)
    si = jnp.imag(S_perm).reshape(_B, _NN)

    fx = jnp.pad(sr, ((0, 0), (0, _NNP - _NN)))
    fy = jnp.pad(si, ((0, 0), (0, _NNP - _NN)))
    sqf = fx * fx + fy * fy
    perms = jnp.asarray(_PERMS_PAD)
    bi = jnp.arange(_B)[:, None]
    ax = jnp.where(jnp.arange(_NAP)[None, :] < _NA, fx[bi, perms], 1e18)
    ay = jnp.where(jnp.arange(_NAP)[None, :] < _NA, fy[bi, perms], 1e18)
    sqa = ax * ax + ay * ay
    am = _knn_topk(_bf(fx), _bf(fy), sqf,
                   _bf(ax)[..., None], _bf(ay)[..., None], sqa[..., None])
    am = jnp.transpose(am, (0, 2, 1))[:, :_NN, :]
    idx = jnp.take_along_axis(perms, am.reshape(_B, -1), axis=1).reshape(_B, _NN, _K)

    ce = theta @ approx
    colh = jnp.asarray(_COLH)

    def wpass(dst, dis, V):
        wd = dis[dst] * dis[:, None] / _S
        fwd = jnp.sum(wd[..., None] * V[dst], axis=1)
        rev = jnp.zeros_like(V).at[dst.reshape(-1)].add(
            (wd[..., None] * V[:, None, :]).reshape(_NN * _K, -1))
        return fwd + rev

    rows = jnp.arange(_NN)
    cmod = rows % _C
    Fs = []
    for b in range(_B):
        dst = idx[b]
        deg = (jnp.zeros((_NN,), jnp.float32).at[dst.reshape(-1)].add(1.0)
               + jnp.float32(_K))
        dis = (deg + 1e-8) ** -0.5
        wd = dis[dst] * dis[:, None] / _S
        dmod = cmod[dst]
        parts = []
        for s in (sr[b], si[b]):
            fwd = jnp.zeros((_NN, _C), jnp.float32).at[
                rows[:, None], dmod].add(wd * s[dst])
            rev = jnp.zeros((_NN, _C), jnp.float32).at[
                dst, cmod[:, None]].add(wd * s[:, None])
            parts.append(fwd + rev)
        P1 = jnp.concatenate(parts, axis=1)
        C0 = jnp.concatenate([sr[b][:, None] * colh, si[b][:, None] * colh], 1)
        U = -ce[1] * C0 + 2.0 * ce[2] * P1
        F = (ce[0] - ce[2]) * C0 + wpass(dst, dis, U)
        Fs.append(F)
    F = jnp.stack(Fs)
    Fr, Fi = F[..., :_C], F[..., _C:]
    Hr = jnp.matmul(Fr, freq_emb, precision=jax.lax.Precision.HIGHEST)
    Hi = jnp.matmul(Fi, freq_emb, precision=jax.lax.Precision.HIGHEST)
    ar = Hr @ Wr_f.T - Hi @ Wi_f.T
    ai = Hr @ Wi_f.T + Hi @ Wr_f.T
    sr_ = jax.nn.silu(ar); si_ = jax.nn.silu(ai)
    zr = (sr_ @ Wr_o.T - si_ @ Wi_o.T)[..., 0]
    zi = (sr_ @ Wi_o.T + si_ @ Wr_o.T)[..., 0]
    zr = zr.reshape(_B, _N, _C); zi = zi.reshape(_B, _N, _C)
    sp = zr @ jnp.asarray(_IDFT_R).T + zi @ jnp.asarray(_IDFT_I).T

    def instnorm(v, g, bb):
        m = jnp.mean(v, -1, keepdims=True); va = jnp.var(v, -1, keepdims=True)
        return g[None, :, None] * (v - m) / jnp.sqrt(va + 1e-5) + bb[None, :, None]

    h = jax.nn.silu(instnorm(sp, g1, b1) @ w1.T + bw1)
    h = jax.nn.silu(instnorm(h, g2, b2) @ w2.T + bw2)
    h = h + (jnp.transpose(trend, (0, 2, 1)) @ wt.T + bt)
    y = h @ w3.T + b3
    return jnp.transpose(y, (0, 2, 1))


# speedup vs baseline: 1.4423x; 1.4423x over previous
import functools

import jax
import jax.numpy as jnp
import numpy as np
from jax.experimental import pallas as pl
from jax.experimental.pallas import tpu as pltpu

_B = 4
_T = 48
_N = 300
_C = 33
_E = 128
_NN = _N * _C
_NA = 5000
_K = 8
_S = 2.0
_SIG = 64

_NNP = 9984
_NAP = 5120
_RT = 128
_ACH = 40

_t = np.arange(_T)
_c = np.arange(_C)
_ang = 2.0 * np.pi * np.outer(_c, _t) / _SIG
_DCT_R = (np.cos(_ang) / np.sqrt(_SIG)).astype(np.float32)
_DCT_I = (-np.sin(_ang) / np.sqrt(_SIG)).astype(np.float32)
_w = np.full(_C, 2.0); _w[0] = 1.0; _w[_C - 1] = 1.0
_ang2 = 2.0 * np.pi * np.outer(_t, _c) / _SIG
_IDFT_R = (_w * np.cos(_ang2) / np.sqrt(_SIG)).astype(np.float32)
_IDFT_I = (-_w * np.sin(_ang2) / np.sqrt(_SIG)).astype(np.float32)

_PERMS = np.stack([np.asarray(jax.random.permutation(
    jax.random.fold_in(jax.random.key(42), b), _NN)[:_NA]) for b in range(_B)])
_PERMS_PAD = np.concatenate(
    [_PERMS, np.zeros((_B, _NAP - _NA), np.int64)], axis=1).astype(np.int32)

_COLH = np.eye(_C, dtype=np.float32)[np.arange(_NN) % _C]


def _knn_body(fxb_ref, fyb_ref, sqf_ref, axb_ref, ayb_ref, sqa_ref,
              out_ref, d2_s, cmin_s):
    fxb = fxb_ref[0, 0]
    fyb = fyb_ref[0, 0]
    sqf = sqf_ref[0, 0]
    iota0 = jax.lax.broadcasted_iota(jnp.int32, (_RT, _RT), 0)
    big_i = jnp.int32(2 ** 30)
    inf = jnp.float32(jnp.inf)

    def fill(c, _):
        sl = pl.ds(c * _RT, _RT)
        a_x = axb_ref[0, sl, :]
        a_y = ayb_ref[0, sl, :]
        s_a = sqa_ref[0, sl, :]
        mm = a_x * fxb + a_y * fyb
        ch = (s_a + sqf) - 2.0 * mm
        d2_s[sl, :] = ch
        cmin_s[pl.ds(c, 1), :] = jnp.min(ch, axis=0, keepdims=True)
        return 0

    jax.lax.fori_loop(0, _ACH, fill, 0)

    for k in range(_K):
        m = jnp.min(cmin_s[...], axis=0, keepdims=True)

        def scan(c, am):
            ch = d2_s[pl.ds(c * _RT, _RT), :]
            cand = jnp.where(ch == m, c * _RT + iota0, big_i)
            return jnp.minimum(am, jnp.min(cand, axis=0, keepdims=True))

        am = jax.lax.fori_loop(0, _ACH, scan,
                               jnp.full((1, _RT), big_i, jnp.int32))

        def mask(c, _):
            sl = pl.ds(c * _RT, _RT)
            ch = d2_s[sl, :]
            ch2 = jnp.where((c * _RT + iota0) == am, inf, ch)
            d2_s[sl, :] = ch2
            cmin_s[pl.ds(c, 1), :] = jnp.min(ch2, axis=0, keepdims=True)
            return 0

        jax.lax.fori_loop(0, _ACH, mask, 0)
        out_ref[0, k, :] = am[0]


@functools.partial(jax.jit, static_argnames=("interpret",))
def _knn_topk(fxb, fyb, sqf, axb, ayb, sqa, interpret=False):
    grid = (_B, _NNP // _RT)
    fxb = fxb.reshape(_B, _NNP // _RT, 1, _RT)
    fyb = fyb.reshape(_B, _NNP // _RT, 1, _RT)
    sqf = sqf.reshape(_B, _NNP // _RT, 1, _RT)
    return pl.pallas_call(
        _knn_body,
        grid=grid,
        in_specs=[
            pl.BlockSpec((1, 1, 1, _RT), lambda b, i: (b, i, 0, 0)),
            pl.BlockSpec((1, 1, 1, _RT), lambda b, i: (b, i, 0, 0)),
            pl.BlockSpec((1, 1, 1, _RT), lambda b, i: (b, i, 0, 0)),
            pl.BlockSpec((1, _NAP, 1), lambda b, i: (b, 0, 0)),
            pl.BlockSpec((1, _NAP, 1), lambda b, i: (b, 0, 0)),
            pl.BlockSpec((1, _NAP, 1), lambda b, i: (b, 0, 0)),
        ],
        out_specs=pl.BlockSpec((1, _K, _RT), lambda b, i: (b, 0, i)),
        out_shape=jax.ShapeDtypeStruct((_B, _K, _NNP), jnp.int32),
        scratch_shapes=[
            pltpu.VMEM((_NAP, _RT), jnp.float32),
            pltpu.VMEM((_ACH, _RT), jnp.float32),
        ],
        interpret=interpret,
    )(fxb, fyb, sqf, axb, ayb, sqa)


def _bf(v):
    return v.astype(jnp.bfloat16).astype(jnp.float32)


def kernel(x, theta, Wr_f, Wi_f, Wr_o, Wi_o, g1, b1, w1, bw1, g2, b2, w2, bw2,
           wt, bt, w3, b3, freq_emb, approx):
    xp = jnp.concatenate([x[:, :1, :], x, x[:, -1:, :]], axis=1)
    trend = (xp[:, :-2, :] + xp[:, 1:-1, :] + xp[:, 2:, :]) / 3.0
    seasonal = x - trend

    Sf = jnp.fft.rfft(seasonal, n=_SIG, axis=1, norm='ortho')
    S_perm = jnp.transpose(Sf, (0, 2, 1))
    sr = jnp.real(S_perm).reshape(_B, _NN)
    si = jnp.imag(S_perm).reshape(_B, _NN)

    fx = jnp.pad(sr, ((0, 0), (0, _NNP - _NN)))
    fy = jnp.pad(si, ((0, 0), (0, _NNP - _NN)))
    sqf = fx * fx + fy * fy
    perms = jnp.asarray(_PERMS_PAD)
    bi = jnp.arange(_B)[:, None]
    ax = jnp.where(jnp.arange(_NAP)[None, :] < _NA, fx[bi, perms], 1e18)
    ay = jnp.where(jnp.arange(_NAP)[None, :] < _NA, fy[bi, perms], 1e18)
    sqa = ax * ax + ay * ay
    am = _knn_topk(_bf(fx), _bf(fy), sqf,
                   _bf(ax)[..., None], _bf(ay)[..., None], sqa[..., None])
    am = jnp.transpose(am, (0, 2, 1))[:, :_NN, :]
    idx = jnp.take_along_axis(perms, am.reshape(_B, -1), axis=1).reshape(_B, _NN, _K)

    ce = theta @ approx
    colh = jnp.asarray(_COLH)

    def wpass(dst, dis, V):
        wd = dis[dst] * dis[:, None] / _S
        fwd = jnp.sum(wd[..., None] * V[dst], axis=1)
        rev = jnp.zeros_like(V).at[dst.reshape(-1)].add(
            (wd[..., None] * V[:, None, :]).reshape(_NN * _K, -1))
        return fwd + rev

    Fs = []
    for b in range(_B):
        dst = idx[b]
        deg = (jnp.zeros((_NN,), jnp.float32).at[dst.reshape(-1)].add(1.0)
               + jnp.float32(_K))
        dis = (deg + 1e-8) ** -0.5
        C0 = jnp.concatenate([sr[b][:, None] * colh, si[b][:, None] * colh], 1)
        P1 = wpass(dst, dis, C0)
        U = -ce[1] * C0 + 2.0 * ce[2] * P1
        F = (ce[0] - ce[2]) * C0 + wpass(dst, dis, U)
        Fs.append(F)
    F = jnp.stack(Fs)
    Fr, Fi = F[..., :_C], F[..., _C:]
    Hr = jnp.matmul(Fr, freq_emb, precision=jax.lax.Precision.HIGHEST)
    Hi = jnp.matmul(Fi, freq_emb, precision=jax.lax.Precision.HIGHEST)
    ar = Hr @ Wr_f.T - Hi @ Wi_f.T
    ai = Hr @ Wi_f.T + Hi @ Wr_f.T
    sr_ = jax.nn.silu(ar); si_ = jax.nn.silu(ai)
    zr = (sr_ @ Wr_o.T - si_ @ Wi_o.T)[..., 0]
    zi = (sr_ @ Wi_o.T + si_ @ Wr_o.T)[..., 0]
    zr = zr.reshape(_B, _N, _C); zi = zi.reshape(_B, _N, _C)
    sp = zr @ jnp.asarray(_IDFT_R).T + zi @ jnp.asarray(_IDFT_I).T

    def instnorm(v, g, bb):
        m = jnp.mean(v, -1, keepdims=True); va = jnp.var(v, -1, keepdims=True)
        return g[None, :, None] * (v - m) / jnp.sqrt(va + 1e-5) + bb[None, :, None]

    h = jax.nn.silu(instnorm(sp, g1, b1) @ w1.T + bw1)
    h = jax.nn.silu(instnorm(h, g2, b2) @ w2.T + bw2)
    h = h + (jnp.transpose(trend, (0, 2, 1)) @ wt.T + bt)
    y = h @ w3.T + b3
    return jnp.transpose(y, (0, 2, 1))
